# Initial kernel scaffold; baseline (speedup 1.0000x reference)
import jax
import jax.numpy as jnp
from jax.experimental import pallas as pl


def kernel(x, mem_kv, W_attn, W_proj, gate_bias):
    raise NotImplementedError("write your pallas kernel here")


# speedup vs baseline: 7.8892x; 7.8892x over previous
import functools

import jax
import jax.numpy as jnp
from jax import lax
from jax.experimental import pallas as pl
from jax.experimental.pallas import tpu as pltpu
from jax.experimental.pallas import tpu_sc as plsc

B, T, C, H, M = 2, 2048, 1024, 16, 8192
DH = C // H
TOPK = 3
BT = B * T

NC, NS = 2, 16
NW = NC * NS

F32 = jnp.float32


_TB_A = 256


def _qkv_body(x_ref, w_ref, q_ref, k_ref, v_ref, kv_ref):
    xx = x_ref[...]
    qkv = jnp.dot(xx, w_ref[...], preferred_element_type=F32)
    q_ref[...] = qkv[:, :C]
    kk = qkv[:, C:2 * C]
    vv = qkv[:, 2 * C:]
    k_ref[...] = kk
    v_ref[...] = vv
    kv_ref[:, 0, :] = kk
    kv_ref[:, 1, :] = vv


def _qkv_call(x2, W_attn):
    grid = (BT // _TB_A,)
    return pl.pallas_call(
        _qkv_body,
        grid=grid,
        in_specs=[
            pl.BlockSpec((_TB_A, C), lambda i: (i, 0)),
            pl.BlockSpec((C, 3 * C), lambda i: (0, 0)),
        ],
        out_specs=[
            pl.BlockSpec((_TB_A, C), lambda i: (i, 0)),
            pl.BlockSpec((_TB_A, C), lambda i: (i, 0)),
            pl.BlockSpec((_TB_A, C), lambda i: (i, 0)),
            pl.BlockSpec((_TB_A, 2, C), lambda i: (i, 0, 0)),
        ],
        out_shape=[
            jax.ShapeDtypeStruct((BT, C), F32),
            jax.ShapeDtypeStruct((BT, C), F32),
            jax.ShapeDtypeStruct((BT, C), F32),
            jax.ShapeDtypeStruct((BT, 2, C), F32),
        ],
    )(x2, W_attn)


_TB_B = 256


def _sdpa_body(q_ref, k_ref, v_ref, y_ref):
    tt = pl.program_id(1)
    q = q_ref[0]
    k = k_ref[0]
    v = v_ref[0]
    s = lax.dot_general(q, k, (((1,), (1,)), ((), ())),
                        preferred_element_type=F32)
    s = s * (1.0 / float(DH) ** 0.5)
    row = tt * _TB_B + lax.broadcasted_iota(jnp.int32, (_TB_B, T), 0)
    col = lax.broadcasted_iota(jnp.int32, (_TB_B, T), 1)
    s = jnp.where(col <= row, s, F32(-1e30))
    m = jnp.max(s, axis=1, keepdims=True)
    p = jnp.exp(s - m)
    l = jnp.sum(p, axis=1, keepdims=True)
    y = jnp.dot(p, v, preferred_element_type=F32)
    y_ref[0] = y / l


def _sdpa_call(qh, kh, vh):
    grid = (B * H, T // _TB_B)
    return pl.pallas_call(
        _sdpa_body,
        grid=grid,
        in_specs=[
            pl.BlockSpec((1, _TB_B, DH), lambda bh, tt: (bh, tt, 0)),
            pl.BlockSpec((1, T, DH), lambda bh, tt: (bh, 0, 0)),
            pl.BlockSpec((1, T, DH), lambda bh, tt: (bh, 0, 0)),
        ],
        out_specs=pl.BlockSpec((1, _TB_B, DH), lambda bh, tt: (bh, tt, 0)),
        out_shape=jax.ShapeDtypeStruct((B * H, T, DH), F32),
    )(qh, kh, vh)


_TB_C = 256
_MB_C = 2048
_NM = M // _MB_C


def _knn_body(q_ref, mk_ref, idx_ref, sims):
    b = pl.program_id(0)
    mm = pl.program_id(2)
    q = q_ref[0]
    mk = mk_ref[0]
    s = lax.dot_general(q, mk, (((1,), (1,)), ((), ())),
                        preferred_element_type=F32)
    sims[:, pl.ds(mm * _MB_C, _MB_C)] = s

    @pl.when(mm == _NM - 1)
    def _():
        sv = sims[...]
        iota = lax.broadcasted_iota(jnp.int32, (_TB_C, M), 1)
        cols = []
        for _k in range(TOPK):
            vmax = jnp.max(sv, axis=1, keepdims=True)
            imax = jnp.min(jnp.where(sv == vmax, iota, M), axis=1,
                           keepdims=True)
            cols.append(imax)
            sv = jnp.where(iota == imax, F32(-jnp.inf), sv)
        gbase = b * M
        cols = [c + gbase for c in cols]
        pad = cols[0]
        idx_ref[0] = jnp.concatenate(cols + [pad] * 5, axis=1)


def _knn_call(q3, mem_keys):
    grid = (B, T // _TB_C, _NM)
    return pl.pallas_call(
        _knn_body,
        grid=grid,
        in_specs=[
            pl.BlockSpec((1, _TB_C, C), lambda b, tt, mm: (b, tt, 0)),
            pl.BlockSpec((1, _MB_C, C), lambda b, tt, mm: (b, mm, 0)),
        ],
        out_specs=pl.BlockSpec((1, _TB_C, 8), lambda b, tt, mm: (b, tt, 0)),
        out_shape=jax.ShapeDtypeStruct((B, T, 8), jnp.int32),
        scratch_shapes=[pltpu.VMEM((_TB_C, M), F32)],
    )(q3, mem_keys)


_NIDX = B * T * TOPK
_ROWS_W = _NIDX // NW
_CHUNK = 24
_NCHUNK = _ROWS_W // _CHUNK


def _gather_body(table_hbm, idx_hbm, out_hbm, idx_v, rows_v, sem):
    wid = lax.axis_index("s") * NC + lax.axis_index("c")
    base = wid * _ROWS_W
    pltpu.sync_copy(idx_hbm.at[pl.ds(base, _ROWS_W)], idx_v)
    for ci in range(_NCHUNK):
        pltpu.async_copy(
            table_hbm.at[idx_v.at[pl.ds(ci * _CHUNK, _CHUNK)]],
            rows_v, sem).wait()
        pltpu.sync_copy(rows_v, out_hbm.at[pl.ds(base + ci * _CHUNK, _CHUNK)])


def _gather_call(table, idx_flat):
    mesh = plsc.VectorSubcoreMesh(core_axis_name="c", subcore_axis_name="s")
    k = pl.kernel(
        _gather_body,
        out_type=jax.ShapeDtypeStruct((_NIDX, 2 * C), F32),
        mesh=mesh,
        scratch_types=[
            pltpu.VMEM((_ROWS_W,), jnp.int32),
            pltpu.VMEM((_CHUNK, 2 * C), F32),
            pltpu.SemaphoreType.DMA,
        ],
    )
    return k(table, idx_flat)


_TB_E = 256


def _combine_body(q_ref, y_ref, g_ref, wp_ref, gate_ref, out_ref):
    q = q_ref[0]
    y = y_ref[0]

    r16 = lax.broadcasted_iota(jnp.int32, (C, H), 0) // DH
    c16 = lax.broadcasted_iota(jnp.int32, (C, H), 1)
    E16 = (r16 == c16).astype(F32)
    r64 = lax.broadcasted_iota(jnp.int32, (H, C), 0)
    c64 = lax.broadcasted_iota(jnp.int32, (H, C), 1) // DH
    E64 = (r64 == c64).astype(F32)

    logits = []
    for kk in range(TOPK):
        gk = g_ref[0, :, kk, :C]
        logits.append(jnp.dot(q * gk, E16, preferred_element_type=F32)
                      * F32(0.125))
    mx = jnp.maximum(jnp.maximum(logits[0], logits[1]), logits[2])
    ws = [jnp.exp(lg - mx) for lg in logits]
    denom = ws[0] + ws[1] + ws[2]
    acc = jnp.zeros((_TB_E, C), F32)
    for kk in range(TOPK):
        gv = g_ref[0, :, kk, C:]
        wexp = jnp.dot(ws[kk], E64, preferred_element_type=F32)
        acc = acc + wexp * gv
    den_exp = jnp.dot(denom, E64, preferred_element_type=F32)
    mem_qkv = acc / den_exp

    gate = gate_ref[0:1, :]
    combined = mem_qkv * gate + y * (1.0 - gate)
    out_ref[0] = jnp.dot(combined, wp_ref[...], preferred_element_type=F32)


def _combine_call(q3, y3, g4, W_proj, gate_row):
    grid = (B, T // _TB_E)
    return pl.pallas_call(
        _combine_body,
        grid=grid,
        in_specs=[
            pl.BlockSpec((1, _TB_E, C), lambda b, tt: (b, tt, 0)),
            pl.BlockSpec((1, _TB_E, C), lambda b, tt: (b, tt, 0)),
            pl.BlockSpec((1, _TB_E, TOPK, 2 * C), lambda b, tt: (b, tt, 0, 0)),
            pl.BlockSpec((C, C), lambda b, tt: (0, 0)),
            pl.BlockSpec((8, C), lambda b, tt: (0, 0)),
        ],
        out_specs=pl.BlockSpec((1, _TB_E, C), lambda b, tt: (b, tt, 0)),
        out_shape=jax.ShapeDtypeStruct((B, T, C), F32),
    )(q3, y3, g4, W_proj, gate_row)


def kernel(x, mem_kv, W_attn, W_proj, gate_bias):
    x2 = x.reshape(BT, C)
    q, k, v, kvmem = _qkv_call(x2, W_attn)

    def to_heads(a):
        return (a.reshape(B, T, H, DH).transpose(0, 2, 1, 3)
                .reshape(B * H, T, DH))

    yh = _sdpa_call(to_heads(q), to_heads(k), to_heads(v))
    y3 = yh.reshape(B, H, T, DH).transpose(0, 2, 1, 3).reshape(B, T, C)

    q3 = q.reshape(B, T, C)
    mem_keys = mem_kv[:, :, 0, :]
    idx8 = _knn_call(q3, mem_keys)
    idx_flat = idx8[:, :, :TOPK].reshape(_NIDX)

    table = mem_kv.reshape(B * M, 2 * C)
    g = _gather_call(table, idx_flat)
    g4 = g.reshape(B, T, TOPK, 2 * C)

    gate_vec = jnp.repeat(gate_bias.reshape(H), DH)
    gate_row = jnp.broadcast_to(gate_vec, (8, C))

    out = _combine_call(q3, y3, g4, W_proj, gate_row)
    kv_memories = kvmem.reshape(B, T, 2, C)
    return out, kv_memories


# speedup vs baseline: 8.4232x; 1.0677x over previous
import functools

import jax
import jax.numpy as jnp
from jax import lax
from jax.experimental import pallas as pl
from jax.experimental.pallas import tpu as pltpu
from jax.experimental.pallas import tpu_sc as plsc

B, T, C, H, M = 2, 2048, 1024, 16, 8192
DH = C // H
TOPK = 3
BT = B * T

NC, NS = 2, 16
NW = NC * NS

F32 = jnp.float32
BF16 = jnp.bfloat16


_TB_A = 256


def _qkv_body(x_ref, w_ref, q_ref, k_ref, v_ref, kv_ref):
    xx = x_ref[...].astype(BF16)
    qkv = jnp.dot(xx, w_ref[...].astype(BF16), preferred_element_type=F32)
    q_ref[...] = qkv[:, :C]
    kk = qkv[:, C:2 * C]
    vv = qkv[:, 2 * C:]
    k_ref[...] = kk
    v_ref[...] = vv
    kv_ref[:, 0, :] = kk
    kv_ref[:, 1, :] = vv


def _qkv_call(x2, W_attn):
    grid = (BT // _TB_A,)
    return pl.pallas_call(
        _qkv_body,
        grid=grid,
        in_specs=[
            pl.BlockSpec((_TB_A, C), lambda i: (i, 0)),
            pl.BlockSpec((C, 3 * C), lambda i: (0, 0)),
        ],
        out_specs=[
            pl.BlockSpec((_TB_A, C), lambda i: (i, 0)),
            pl.BlockSpec((_TB_A, C), lambda i: (i, 0)),
            pl.BlockSpec((_TB_A, C), lambda i: (i, 0)),
            pl.BlockSpec((_TB_A, 2, C), lambda i: (i, 0, 0)),
        ],
        out_shape=[
            jax.ShapeDtypeStruct((BT, C), F32),
            jax.ShapeDtypeStruct((BT, C), F32),
            jax.ShapeDtypeStruct((BT, C), F32),
            jax.ShapeDtypeStruct((BT, 2, C), F32),
        ],
    )(x2, W_attn)


_TB_B = 256


def _sdpa_body(q_ref, k_ref, v_ref, y_ref):
    tt = pl.program_id(1)
    q = q_ref[0].astype(BF16)
    k = k_ref[0].astype(BF16)
    v = v_ref[0].astype(BF16)
    s = lax.dot_general(q, k, (((1,), (1,)), ((), ())),
                        preferred_element_type=F32)
    s = s * (1.0 / float(DH) ** 0.5)
    row = tt * _TB_B + lax.broadcasted_iota(jnp.int32, (_TB_B, T), 0)
    col = lax.broadcasted_iota(jnp.int32, (_TB_B, T), 1)
    s = jnp.where(col <= row, s, F32(-1e30))
    m = jnp.max(s, axis=1, keepdims=True)
    p = jnp.exp(s - m)
    l = jnp.sum(p, axis=1, keepdims=True)
    y = jnp.dot(p.astype(BF16), v, preferred_element_type=F32)
    y_ref[0] = y / l


def _sdpa_call(qh, kh, vh):
    grid = (B * H, T // _TB_B)
    return pl.pallas_call(
        _sdpa_body,
        grid=grid,
        in_specs=[
            pl.BlockSpec((1, _TB_B, DH), lambda bh, tt: (bh, tt, 0)),
            pl.BlockSpec((1, T, DH), lambda bh, tt: (bh, 0, 0)),
            pl.BlockSpec((1, T, DH), lambda bh, tt: (bh, 0, 0)),
        ],
        out_specs=pl.BlockSpec((1, _TB_B, DH), lambda bh, tt: (bh, tt, 0)),
        out_shape=jax.ShapeDtypeStruct((B * H, T, DH), F32),
    )(qh, kh, vh)


_TB_C = 256
_MB_C = 2048
_NM = M // _MB_C


def _knn_body(q_ref, mk_ref, idx_ref, sims):
    b = pl.program_id(0)
    mm = pl.program_id(2)
    q = q_ref[0].astype(BF16)
    mk = mk_ref[0].astype(BF16)
    s = lax.dot_general(q, mk, (((1,), (1,)), ((), ())),
                        preferred_element_type=F32)
    sims[:, pl.ds(mm * _MB_C, _MB_C)] = s

    @pl.when(mm == _NM - 1)
    def _():
        sv = sims[...]
        iota = lax.broadcasted_iota(jnp.int32, (_TB_C, M), 1)
        cols = []
        for _k in range(TOPK):
            vmax = jnp.max(sv, axis=1, keepdims=True)
            imax = jnp.min(jnp.where(sv == vmax, iota, M), axis=1,
                           keepdims=True)
            cols.append(imax)
            sv = jnp.where(iota == imax, F32(-jnp.inf), sv)
        gbase = b * M
        cols = [c + gbase for c in cols]
        pad = cols[0]
        idx_ref[0] = jnp.concatenate(cols + [pad] * 5, axis=1)


def _knn_call(q3, mem_keys):
    grid = (B, T // _TB_C, _NM)
    return pl.pallas_call(
        _knn_body,
        grid=grid,
        in_specs=[
            pl.BlockSpec((1, _TB_C, C), lambda b, tt, mm: (b, tt, 0)),
            pl.BlockSpec((1, _MB_C, C), lambda b, tt, mm: (b, mm, 0)),
        ],
        out_specs=pl.BlockSpec((1, _TB_C, 8), lambda b, tt, mm: (b, tt, 0)),
        out_shape=jax.ShapeDtypeStruct((B, T, 8), jnp.int32),
        scratch_shapes=[pltpu.VMEM((_TB_C, M), F32)],
    )(q3, mem_keys)


_NIDX = B * T * TOPK
_ROWS_W = _NIDX // NW
_CHUNK = 24
_NCHUNK = _ROWS_W // _CHUNK


def _gather_body(table_hbm, idx_hbm, out_hbm, idx_v, rows_v, sem):
    wid = lax.axis_index("s") * NC + lax.axis_index("c")
    base = wid * _ROWS_W
    pltpu.sync_copy(idx_hbm.at[pl.ds(base, _ROWS_W)], idx_v)
    for ci in range(_NCHUNK):
        pltpu.async_copy(
            table_hbm.at[idx_v.at[pl.ds(ci * _CHUNK, _CHUNK)]],
            rows_v, sem).wait()
        pltpu.sync_copy(rows_v, out_hbm.at[pl.ds(base + ci * _CHUNK, _CHUNK)])


def _gather_call(table, idx_flat):
    mesh = plsc.VectorSubcoreMesh(core_axis_name="c", subcore_axis_name="s")
    k = pl.kernel(
        _gather_body,
        out_type=jax.ShapeDtypeStruct((_NIDX, 2 * C), F32),
        mesh=mesh,
        scratch_types=[
            pltpu.VMEM((_ROWS_W,), jnp.int32),
            pltpu.VMEM((_CHUNK, 2 * C), F32),
            pltpu.SemaphoreType.DMA,
        ],
    )
    return k(table, idx_flat)


_TB_E = 256


def _combine_body(q_ref, y_ref, g_ref, wp_ref, gate_ref, out_ref):
    q = q_ref[0]
    y = y_ref[0]

    r16 = lax.broadcasted_iota(jnp.int32, (C, H), 0) // DH
    c16 = lax.broadcasted_iota(jnp.int32, (C, H), 1)
    E16 = (r16 == c16).astype(BF16)
    r64 = lax.broadcasted_iota(jnp.int32, (H, C), 0)
    c64 = lax.broadcasted_iota(jnp.int32, (H, C), 1) // DH
    E64 = (r64 == c64).astype(BF16)

    logits = []
    for kk in range(TOPK):
        gk = g_ref[0, :, kk, :C]
        logits.append(jnp.dot((q * gk).astype(BF16), E16,
                              preferred_element_type=F32)
                      * F32(0.125))
    mx = jnp.maximum(jnp.maximum(logits[0], logits[1]), logits[2])
    ws = [jnp.exp(lg - mx) for lg in logits]
    denom = ws[0] + ws[1] + ws[2]
    acc = jnp.zeros((_TB_E, C), F32)
    for kk in range(TOPK):
        gv = g_ref[0, :, kk, C:]
        wexp = jnp.dot(ws[kk].astype(BF16), E64, preferred_element_type=F32)
        acc = acc + wexp * gv
    den_exp = jnp.dot(denom.astype(BF16), E64, preferred_element_type=F32)
    mem_qkv = acc / den_exp

    gate = gate_ref[0:1, :]
    combined = mem_qkv * gate + y * (1.0 - gate)
    out_ref[0] = jnp.dot(combined.astype(BF16), wp_ref[...].astype(BF16),
                         preferred_element_type=F32)


def _combine_call(q3, y3, g4, W_proj, gate_row):
    grid = (B, T // _TB_E)
    return pl.pallas_call(
        _combine_body,
        grid=grid,
        in_specs=[
            pl.BlockSpec((1, _TB_E, C), lambda b, tt: (b, tt, 0)),
            pl.BlockSpec((1, _TB_E, C), lambda b, tt: (b, tt, 0)),
            pl.BlockSpec((1, _TB_E, TOPK, 2 * C), lambda b, tt: (b, tt, 0, 0)),
            pl.BlockSpec((C, C), lambda b, tt: (0, 0)),
            pl.BlockSpec((8, C), lambda b, tt: (0, 0)),
        ],
        out_specs=pl.BlockSpec((1, _TB_E, C), lambda b, tt: (b, tt, 0)),
        out_shape=jax.ShapeDtypeStruct((B, T, C), F32),
    )(q3, y3, g4, W_proj, gate_row)


def kernel(x, mem_kv, W_attn, W_proj, gate_bias):
    x2 = x.reshape(BT, C)
    q, k, v, kvmem = _qkv_call(x2, W_attn)

    def to_heads(a):
        return (a.reshape(B, T, H, DH).transpose(0, 2, 1, 3)
                .reshape(B * H, T, DH))

    yh = _sdpa_call(to_heads(q), to_heads(k), to_heads(v))
    y3 = yh.reshape(B, H, T, DH).transpose(0, 2, 1, 3).reshape(B, T, C)

    q3 = q.reshape(B, T, C)
    mem_keys = mem_kv[:, :, 0, :]
    idx8 = _knn_call(q3, mem_keys)
    idx_flat = idx8[:, :, :TOPK].reshape(_NIDX)

    table = mem_kv.reshape(B * M, 2 * C)
    g = _gather_call(table, idx_flat)
    g4 = g.reshape(B, T, TOPK, 2 * C)

    gate_vec = jnp.repeat(gate_bias.reshape(H), DH)
    gate_row = jnp.broadcast_to(gate_vec, (8, C))

    out = _combine_call(q3, y3, g4, W_proj, gate_row)
    kv_memories = kvmem.reshape(B, T, 2, C)
    return out, kv_memories
